# baseline (speedup 1.0000x reference)
import functools

import jax
import jax.numpy as jnp
from jax import lax
from jax.experimental import pallas as pl
from jax.experimental.pallas import tpu as pltpu
from jax.experimental.pallas import tpu_sc as plsc

NUM_ROWS = 16384
NUM_COLS = 200
EMB = 3
N_IDX = NUM_ROWS * NUM_COLS
N_OUT = N_IDX * EMB

NC = 2
NS = 16
NW = NC * NS
LANES = 16

PER_W = N_IDX // NW
CHUNK = 10_240
NCH = PER_W // CHUNK
NITER = CHUNK // LANES


def _sc_body(arg_hbm, w_hbm, out_hbm,
             idx0, idx1, ob0, ob1, wtab,
             sem_i0, sem_i1, sem_o0, sem_o1):
    wid = lax.axis_index("s") * NC + lax.axis_index("c")
    ibase = wid * PER_W

    pltpu.sync_copy(w_hbm, wtab)

    idx_bufs = (idx0, idx1)
    out_bufs = (ob0, ob1)
    sems_i = (sem_i0, sem_i1)
    sems_o = (sem_o0, sem_o1)

    def start_idx(c):
        return pltpu.async_copy(
            arg_hbm.at[pl.ds(ibase + c * CHUNK, CHUNK)],
            idx_bufs[c % 2], sems_i[c % 2])

    def compute(idx_ref, out_ref):
        @plsc.parallel_loop(0, NITER, 1, unroll=8)
        def _(i):
            av = idx_ref[pl.ds(i * LANES, LANES)]
            b3 = av * 3
            for k in range(EMB):
                g = plsc.load_gather(wtab, [b3 + k])
                out_ref[pl.ds(k * CHUNK + i * LANES, LANES)] = g

    h_out = [None, None]
    h_idx = start_idx(0)
    for c in range(NCH):
        h_next = start_idx(c + 1) if c + 1 < NCH else None
        h_idx.wait()
        if h_out[c % 2] is not None:
            for h in h_out[c % 2]:
                h.wait()
        compute(idx_bufs[c % 2], out_bufs[c % 2])
        h_out[c % 2] = [
            pltpu.async_copy(
                out_bufs[c % 2].at[pl.ds(k * CHUNK, CHUNK)],
                out_hbm.at[pl.ds(k * N_IDX + ibase + c * CHUNK, CHUNK)],
                sems_o[c % 2])
            for k in range(EMB)
        ]
        h_idx = h_next
    for hs in h_out:
        if hs is not None:
            for h in hs:
                h.wait()


@functools.partial(jax.jit, static_argnames=())
def _sc_lookup(arg_flat, w_pad):
    mesh = plsc.VectorSubcoreMesh(core_axis_name="c", subcore_axis_name="s")
    f = pl.kernel(
        _sc_body,
        out_type=jax.ShapeDtypeStruct((N_OUT,), jnp.float32),
        mesh=mesh,
        scratch_types=[
            pltpu.VMEM((CHUNK,), jnp.int32),
            pltpu.VMEM((CHUNK,), jnp.int32),
            pltpu.VMEM((CHUNK * EMB,), jnp.float32),
            pltpu.VMEM((CHUNK * EMB,), jnp.float32),
            pltpu.VMEM((32,), jnp.float32),
            pltpu.SemaphoreType.DMA,
            pltpu.SemaphoreType.DMA,
            pltpu.SemaphoreType.DMA,
            pltpu.SemaphoreType.DMA,
        ],
        compiler_params=pltpu.CompilerParams(needs_layout_passes=False),
    )
    return f(arg_flat, w_pad)


def kernel(arg, weight):
    a_flat = (arg.T.astype(jnp.int32)
              .reshape(25, 8, 128, 128)
              .transpose(0, 2, 1, 3)
              .reshape(N_IDX))
    w_pad = jnp.pad(weight.reshape(-1), (0, 2))
    o = _sc_lookup(a_flat, w_pad)
    o5 = (o.reshape(EMB, 25, 128, 8, 128)
          .transpose(0, 1, 3, 2, 4)
          .reshape(EMB, NUM_COLS, NUM_ROWS))
    return o5.transpose(2, 1, 0)


# speedup vs baseline: 1.0022x; 1.0022x over previous
import functools

import jax
import jax.numpy as jnp
from jax import lax
from jax.experimental import pallas as pl
from jax.experimental.pallas import tpu as pltpu
from jax.experimental.pallas import tpu_sc as plsc

NUM_ROWS = 16384
NUM_COLS = 200
EMB = 3
N_IDX = NUM_ROWS * NUM_COLS
N_OUT = N_IDX * EMB

NC = 2
NS = 16
NW = NC * NS
LANES = 16

PER_W = N_IDX // NW
CHUNK = 10_240
NCH = PER_W // CHUNK
NITER = CHUNK // LANES


def _sc_body(arg_hbm, w_hbm, out_hbm,
             idx0, idx1, ob0, ob1, wtab,
             sem_i0, sem_i1, sem_o0, sem_o1):
    wid = lax.axis_index("s") * NC + lax.axis_index("c")
    ibase = wid * PER_W

    pltpu.sync_copy(w_hbm, wtab)

    idx_bufs = (idx0, idx1)
    out_bufs = (ob0, ob1)
    sems_i = (sem_i0, sem_i1)
    sems_o = (sem_o0, sem_o1)

    def start_idx(c):
        return pltpu.async_copy(
            arg_hbm.at[pl.ds(ibase + c * CHUNK, CHUNK)],
            idx_bufs[c % 2], sems_i[c % 2])

    def compute(idx_ref, out_ref):
        @plsc.parallel_loop(0, NITER, 1, unroll=8)
        def _(i):
            av = idx_ref[pl.ds(i * LANES, LANES)]
            b3 = av * 3
            for k in range(EMB):
                g = plsc.load_gather(wtab, [b3 + k])
                out_ref[pl.ds(k * CHUNK + i * LANES, LANES)] = g

    h_out = [None, None]
    h_idx = start_idx(0)
    for c in range(NCH):
        h_next = start_idx(c + 1) if c + 1 < NCH else None
        h_idx.wait()
        if h_out[c % 2] is not None:
            for h in h_out[c % 2]:
                h.wait()
        compute(idx_bufs[c % 2], out_bufs[c % 2])
        h_out[c % 2] = [
            pltpu.async_copy(
                out_bufs[c % 2].at[pl.ds(k * CHUNK, CHUNK)],
                out_hbm.at[pl.ds(k * N_IDX + ibase + c * CHUNK, CHUNK)],
                sems_o[c % 2])
            for k in range(EMB)
        ]
        h_idx = h_next
    for hs in h_out:
        if hs is not None:
            for h in hs:
                h.wait()


@functools.partial(jax.jit, static_argnames=())
def _sc_lookup(arg_flat, w_pad):
    mesh = plsc.VectorSubcoreMesh(core_axis_name="c", subcore_axis_name="s")
    f = pl.kernel(
        _sc_body,
        out_type=jax.ShapeDtypeStruct((N_OUT,), jnp.float32),
        mesh=mesh,
        scratch_types=[
            pltpu.VMEM((CHUNK,), jnp.int32),
            pltpu.VMEM((CHUNK,), jnp.int32),
            pltpu.VMEM((CHUNK * EMB,), jnp.float32),
            pltpu.VMEM((CHUNK * EMB,), jnp.float32),
            pltpu.VMEM((30,), jnp.float32),
            pltpu.SemaphoreType.DMA,
            pltpu.SemaphoreType.DMA,
            pltpu.SemaphoreType.DMA,
            pltpu.SemaphoreType.DMA,
        ],
        compiler_params=pltpu.CompilerParams(needs_layout_passes=False),
    )
    return f(arg_flat, w_pad)


def kernel(arg, weight):
    a_flat = (arg.T.astype(jnp.int32)
              .reshape(25, 8, 128, 128)
              .transpose(0, 2, 1, 3)
              .reshape(N_IDX))
    o = _sc_lookup(a_flat, weight.reshape(-1))
    o5 = (o.reshape(EMB, 25, 128, 8, 128)
          .transpose(0, 1, 3, 2, 4)
          .reshape(EMB, NUM_COLS, NUM_ROWS))
    return o5.transpose(2, 1, 0)


# speedup vs baseline: 1.0113x; 1.0091x over previous
import functools

import jax
import jax.numpy as jnp
from jax import lax
from jax.experimental import pallas as pl
from jax.experimental.pallas import tpu as pltpu
from jax.experimental.pallas import tpu_sc as plsc

NUM_ROWS = 16384
NUM_COLS = 200
EMB = 3
N_IDX = NUM_ROWS * NUM_COLS
N_OUT = N_IDX * EMB

NC = 2
NS = 16
NW = NC * NS
LANES = 16

PER_W = N_IDX // NW
CHUNK = 12_800
NCH = PER_W // CHUNK
NITER = CHUNK // LANES


def _sc_body(arg_hbm, w_hbm, out_hbm,
             idx0, idx1, ob0, ob1, wtab,
             sem_i0, sem_i1, sem_o0, sem_o1):
    wid = lax.axis_index("s") * NC + lax.axis_index("c")
    ibase = wid * PER_W

    pltpu.sync_copy(w_hbm, wtab)

    idx_bufs = (idx0, idx1)
    out_bufs = (ob0, ob1)
    sems_i = (sem_i0, sem_i1)
    sems_o = (sem_o0, sem_o1)

    def start_idx(c):
        return pltpu.async_copy(
            arg_hbm.at[pl.ds(ibase + c * CHUNK, CHUNK)],
            idx_bufs[c % 2], sems_i[c % 2])

    def compute(idx_ref, out_ref):
        @plsc.parallel_loop(0, NITER, 1, unroll=8)
        def _(i):
            av = idx_ref[pl.ds(i * LANES, LANES)]
            b3 = av * 3
            for k in range(EMB):
                g = plsc.load_gather(wtab, [b3 + k])
                out_ref[pl.ds(k * CHUNK + i * LANES, LANES)] = g

    h_out = [None, None]
    h_idx = start_idx(0)
    for c in range(NCH):
        h_next = start_idx(c + 1) if c + 1 < NCH else None
        h_idx.wait()
        if h_out[c % 2] is not None:
            for h in h_out[c % 2]:
                h.wait()
        compute(idx_bufs[c % 2], out_bufs[c % 2])
        h_out[c % 2] = [
            pltpu.async_copy(
                out_bufs[c % 2].at[pl.ds(k * CHUNK, CHUNK)],
                out_hbm.at[pl.ds(k * N_IDX + ibase + c * CHUNK, CHUNK)],
                sems_o[c % 2])
            for k in range(EMB)
        ]
        h_idx = h_next
    for hs in h_out:
        if hs is not None:
            for h in hs:
                h.wait()


@functools.partial(jax.jit, static_argnames=())
def _sc_lookup(arg_flat, w_pad):
    mesh = plsc.VectorSubcoreMesh(core_axis_name="c", subcore_axis_name="s")
    f = pl.kernel(
        _sc_body,
        out_type=jax.ShapeDtypeStruct((N_OUT,), jnp.float32),
        mesh=mesh,
        scratch_types=[
            pltpu.VMEM((CHUNK,), jnp.int32),
            pltpu.VMEM((CHUNK,), jnp.int32),
            pltpu.VMEM((CHUNK * EMB,), jnp.float32),
            pltpu.VMEM((CHUNK * EMB,), jnp.float32),
            pltpu.VMEM((30,), jnp.float32),
            pltpu.SemaphoreType.DMA,
            pltpu.SemaphoreType.DMA,
            pltpu.SemaphoreType.DMA,
            pltpu.SemaphoreType.DMA,
        ],
        compiler_params=pltpu.CompilerParams(needs_layout_passes=False),
    )
    return f(arg_flat, w_pad)


def kernel(arg, weight):
    a_flat = (arg.T.astype(jnp.int32)
              .reshape(25, 8, 128, 128)
              .transpose(0, 2, 1, 3)
              .reshape(N_IDX))
    o = _sc_lookup(a_flat, weight.reshape(-1))
    o5 = (o.reshape(EMB, 25, 128, 8, 128)
          .transpose(0, 1, 3, 2, 4)
          .reshape(EMB, NUM_COLS, NUM_ROWS))
    return o5.transpose(2, 1, 0)
